# baseline (speedup 1.0000x reference)
import dataclasses
import functools

import jax
import jax.numpy as jnp
import numpy as np
from jax import lax
from jax.experimental import pallas as pl
from jax.experimental.pallas import tpu as pltpu
from jax.experimental.pallas import tpu_sc as plsc

B = 16384
L = 50
LP = 64
D = 64
ALPHA = 0.1

NW = 32
NSUB = 16
SPW = B // NW
PASSES = 4
SPP = SPW // PASSES
SC = 2
K = SC * LP
CPP = SPP // SC
NLANE = 16
DEPTH = 4
DUMP = NSUB * SPP
ACC_ROWS = NSUB * SPP + 8


def _sess_table():
    t = np.empty((NSUB, CPP, K), np.int32)
    for s in range(NSUB):
        for c in range(CPP):
            for r in range(SC):
                t[s, c, r * LP:r * LP + L] = s * SPP + c * SC + r
                t[s, c, r * LP + L:(r + 1) * LP] = DUMP
    return t


def _build():
    mesh = plsc.VectorSubcoreMesh(core_axis_name="c", subcore_axis_name="s")
    cp = pltpu.CompilerParams(use_tc_tiling_on_sc=False)
    if "needs_layout_passes" in pltpu.CompilerParams.__dataclass_fields__:
        cp = dataclasses.replace(cp, needs_layout_passes=False)

    scratch = [
        pltpu.VMEM((SPP, LP), jnp.int32),
        pltpu.VMEM((CPP, K), jnp.int32),
    ]
    scratch += [pltpu.VMEM((LP,), jnp.int32) for _ in range(SC * DEPTH)]
    scratch += [pltpu.VMEM((K, D), jnp.float32) for _ in range(DEPTH)]
    scratch += [pltpu.VMEM((K, D), jnp.float32) for _ in range(DEPTH)]
    scratch += [
        pltpu.VMEM((SPP, D), jnp.float32),
        pltpu.VMEM((SPP, D), jnp.float32),
        pltpu.VMEM_SHARED((ACC_ROWS, D), jnp.float32),
        pltpu.VMEM_SHARED((ACC_ROWS, D), jnp.float32),
    ]
    scratch += [pltpu.SemaphoreType.DMA] * (4 * DEPTH)

    @functools.partial(
        pl.kernel,
        out_type=jax.ShapeDtypeStruct((B, D), jnp.float32),
        mesh=mesh,
        compiler_params=cp,
        scratch_types=scratch,
    )
    def k(idx_hbm, item_hbm, cat_hbm, i2c_hbm, sess_hbm, out_hbm,
          idx_slab, sess_vm, *rest):
        cidx = rest[0:SC * DEPTH]
        ibuf = rest[SC * DEPTH:SC * DEPTH + DEPTH]
        cbuf = rest[SC * DEPTH + DEPTH:SC * DEPTH + 2 * DEPTH]
        icomb, ccomb, iacc, cacc = rest[SC * DEPTH + 2 * DEPTH:
                                        SC * DEPTH + 2 * DEPTH + 4]
        sems = rest[SC * DEPTH + 2 * DEPTH + 4:]
        semL = sems[0:DEPTH]
        semI = sems[DEPTH:2 * DEPTH]
        semC = sems[2 * DEPTH:3 * DEPTH]
        semS = sems[3 * DEPTH:4 * DEPTH]

        sid = lax.axis_index("s")
        wid = sid * 2 + lax.axis_index("c")

        zero = jnp.zeros((NLANE,), jnp.float32)
        inv_l = jnp.float32(1.0 / L)
        alpha = jnp.float32(ALPHA)
        my_rows = pl.ds(sid * SPP, SPP)

        pltpu.sync_copy(sess_hbm.at[sid], sess_vm)

        def gather_pair(r, j):
            for q in range(SC):
                row = idx_slab.at[SC * r + q]
                pltpu.async_copy(i2c_hbm.at[row], cidx[SC * j + q], semL[j])
                pltpu.async_copy(item_hbm.at[row],
                                 ibuf[j].at[pl.ds(q * LP, LP)], semI[j])

        def cat_stage(r, j):
            for q in range(SC):
                pltpu.make_async_copy(i2c_hbm.at[idx_slab.at[0]],
                                      cidx[SC * j + q], semL[j]).wait()
            for q in range(SC):
                pltpu.async_copy(cat_hbm.at[cidx[SC * j + q]],
                                 cbuf[j].at[pl.ds(q * LP, LP)], semC[j])

        def scatter_stage(r, j):
            for q in range(SC):
                pltpu.make_async_copy(item_hbm.at[idx_slab.at[0]],
                                      ibuf[j].at[pl.ds(q * LP, LP)],
                                      semI[j]).wait()
                pltpu.make_async_copy(cat_hbm.at[cidx[SC * j + q]],
                                      cbuf[j].at[pl.ds(q * LP, LP)],
                                      semC[j]).wait()
            sref = sess_vm.at[r]
            pltpu.async_copy(ibuf[j], iacc.at[sref], semS[j], add=True)
            pltpu.async_copy(cbuf[j], cacc.at[sref], semS[j], add=True)

        def sa_drain(r, j):
            sref = sess_vm.at[r]
            pltpu.make_async_copy(ibuf[j], iacc.at[sref], semS[j]).wait()
            pltpu.make_async_copy(cbuf[j], cacc.at[sref], semS[j]).wait()

        for h in range(PASSES):
            pltpu.sync_copy(idx_hbm.at[pl.ds(wid * SPW + h * SPP, SPP)],
                            idx_slab)

            @pl.loop(0, SPP)
            def _(s):
                for d in range(D // NLANE):
                    icomb.at[s, pl.ds(d * NLANE, NLANE)][...] = zero

            pltpu.sync_copy(icomb, iacc.at[my_rows])
            pltpu.sync_copy(icomb, cacc.at[my_rows])

            gather_pair(0, 0)
            gather_pair(1, 1)
            cat_stage(0, 0)

            @pl.loop(0, CPP // DEPTH)
            def _(t):
                for j in range(DEPTH):
                    r = DEPTH * t + j
                    sj2 = (j + 2) % DEPTH
                    sj1 = (j + 1) % DEPTH

                    @pl.when(r >= DEPTH - 2)
                    def _(r=r, sj2=sj2):
                        sa_drain(r - 2, sj2)

                    @pl.when(r <= CPP - 3)
                    def _(r=r, sj2=sj2):
                        gather_pair(r + 2, sj2)

                    @pl.when(r <= CPP - 2)
                    def _(r=r, sj1=sj1):
                        cat_stage(r + 1, sj1)

                    scatter_stage(r, j)

            for tail in range(DEPTH - 2, 0, -1):
                sa_drain(CPP - tail, (CPP - tail) % DEPTH)

            pltpu.sync_copy(iacc.at[my_rows], icomb)
            pltpu.sync_copy(cacc.at[my_rows], ccomb)

            @pl.loop(0, SPP)
            def _(s):
                for d in range(D // NLANE):
                    slc = (s, pl.ds(d * NLANE, NLANE))
                    a = icomb.at[slc][...]
                    b = ccomb.at[slc][...]
                    icomb.at[slc][...] = (a + alpha * b) * inv_l

            pltpu.sync_copy(icomb, out_hbm.at[pl.ds(wid * SPW + h * SPP, SPP)])

    return k


_k = _build()
_SESS = _sess_table()


def kernel(indices, item_table, cat_table, item_to_cat):
    idx_p = jnp.pad(indices, ((0, 0), (0, LP - L)))
    return _k(idx_p, item_table, cat_table, item_to_cat, _SESS)


# speedup vs baseline: 5.0465x; 5.0465x over previous
import dataclasses
import functools

import jax
import jax.numpy as jnp
from jax import lax
from jax.experimental import pallas as pl
from jax.experimental.pallas import tpu as pltpu
from jax.experimental.pallas import tpu_sc as plsc

B = 16384
L = 50
D = 64
NCAT = 1000
ALPHA = 0.1

NW = 32
NSUB = 16
TPW = B * L // NW
K = 128
NCHUNK = TPW // K
SPW = B // NW
PASSES = 4
SPP = SPW // PASSES
CPP = NCHUNK // PASSES
NLANE = 16
DEPTH = 5


def _build():
    mesh = plsc.VectorSubcoreMesh(core_axis_name="c", subcore_axis_name="s")
    cp = pltpu.CompilerParams(use_tc_tiling_on_sc=False)
    if "needs_layout_passes" in pltpu.CompilerParams.__dataclass_fields__:
        cp = dataclasses.replace(cp, needs_layout_passes=False)

    scratch = [pltpu.VMEM((CPP, K), jnp.int32)]
    scratch += [pltpu.VMEM((K,), jnp.int32) for _ in range(DEPTH)]
    scratch += [pltpu.VMEM((K,), jnp.int32) for _ in range(DEPTH)]
    scratch += [pltpu.VMEM((K, D), jnp.float32) for _ in range(DEPTH)]
    scratch += [pltpu.VMEM((K, D), jnp.float32) for _ in range(DEPTH)]
    scratch += [
        pltpu.VMEM((SPP, D), jnp.float32),
        pltpu.VMEM((SPP, D), jnp.float32),
        pltpu.VMEM_SHARED((NSUB * SPP, D), jnp.float32),
        pltpu.VMEM_SHARED((NSUB * SPP, D), jnp.float32),
        pltpu.VMEM_SHARED((NCAT, D), jnp.float32),
    ]
    scratch += [pltpu.SemaphoreType.DMA] * (4 * DEPTH)

    @functools.partial(
        pl.kernel,
        out_type=jax.ShapeDtypeStruct((B, D), jnp.float32),
        mesh=mesh,
        compiler_params=cp,
        scratch_types=scratch,
    )
    def k(idx_hbm, item_hbm, cat_hbm, i2c_hbm, out_hbm, idx_slab, *rest):
        cidx = rest[0:DEPTH]
        sess = rest[DEPTH:2 * DEPTH]
        ibuf = rest[2 * DEPTH:3 * DEPTH]
        cbuf = rest[3 * DEPTH:4 * DEPTH]
        icomb, ccomb, iacc, cacc, scat = rest[4 * DEPTH:4 * DEPTH + 5]
        sems = rest[4 * DEPTH + 5:]
        semL = sems[0:DEPTH]
        semI = sems[DEPTH:2 * DEPTH]
        semC = sems[2 * DEPTH:3 * DEPTH]
        semS = sems[3 * DEPTH:4 * DEPTH]

        sid = lax.axis_index("s")
        wid = sid * 2 + lax.axis_index("c")

        zero = jnp.zeros((NLANE,), jnp.float32)
        iota = lax.iota(jnp.int32, NLANE)
        inv_l = jnp.float32(1.0 / L)
        alpha = jnp.float32(ALPHA)
        my_rows = pl.ds(sid * SPP, SPP)

        @pl.when(sid == 0)
        def _():
            pltpu.sync_copy(cat_hbm, scat)

        plsc.subcore_barrier()

        def gather_pair(r, j):
            idx_row = idx_slab.at[r]
            pltpu.async_copy(i2c_hbm.at[idx_row], cidx[j], semL[j])
            pltpu.async_copy(item_hbm.at[idx_row], ibuf[j], semI[j])

        def cat_stage(r, j, h):
            for g in range(K // NLANE):
                tok = iota + ((h * CPP + r) * K + g * NLANE)
                sess[j].at[pl.ds(g * NLANE, NLANE)][...] = (
                    sid * SPP + (tok // L - h * SPP))
            pltpu.make_async_copy(i2c_hbm.at[idx_slab.at[r]], cidx[j],
                                  semL[j]).wait()
            pltpu.async_copy(scat.at[cidx[j]], cbuf[j], semC[j])

        def scatter_stage(r, j):
            pltpu.make_async_copy(item_hbm.at[idx_slab.at[r]], ibuf[j],
                                  semI[j]).wait()
            pltpu.make_async_copy(scat.at[cidx[j]], cbuf[j], semC[j]).wait()
            pltpu.async_copy(ibuf[j], iacc.at[sess[j]], semS[j], add=True)
            pltpu.async_copy(cbuf[j], cacc.at[sess[j]], semS[j], add=True)

        def sa_drain(j):
            pltpu.make_async_copy(ibuf[j], iacc.at[sess[j]], semS[j]).wait()
            pltpu.make_async_copy(cbuf[j], cacc.at[sess[j]], semS[j]).wait()

        for h in range(PASSES):
            pltpu.sync_copy(idx_hbm.at[wid, pl.ds(h * CPP, CPP)], idx_slab)

            @pl.loop(0, SPP)
            def _(s):
                for d in range(D // NLANE):
                    icomb.at[s, pl.ds(d * NLANE, NLANE)][...] = zero

            pltpu.sync_copy(icomb, iacc.at[my_rows])
            pltpu.sync_copy(icomb, cacc.at[my_rows])

            gather_pair(0, 0)
            gather_pair(1, 1)
            cat_stage(0, 0, h)

            @pl.loop(0, CPP // DEPTH)
            def _(t):
                for j in range(DEPTH):
                    r = DEPTH * t + j
                    sj2 = (j + 2) % DEPTH
                    sj1 = (j + 1) % DEPTH

                    @pl.when(r >= DEPTH - 2)
                    def _(sj2=sj2):
                        sa_drain(sj2)

                    @pl.when(r <= CPP - 3)
                    def _(r=r, sj2=sj2):
                        gather_pair(r + 2, sj2)

                    @pl.when(r <= CPP - 2)
                    def _(r=r, sj1=sj1):
                        cat_stage(r + 1, sj1, h)

                    scatter_stage(r, j)

            for tail in range(DEPTH - 2, 0, -1):
                sa_drain((CPP - tail) % DEPTH)

            pltpu.sync_copy(iacc.at[my_rows], icomb)
            pltpu.sync_copy(cacc.at[my_rows], ccomb)

            @pl.loop(0, SPP)
            def _(s):
                for d in range(D // NLANE):
                    slc = (s, pl.ds(d * NLANE, NLANE))
                    a = icomb.at[slc][...]
                    b = ccomb.at[slc][...]
                    icomb.at[slc][...] = (a + alpha * b) * inv_l

            pltpu.sync_copy(icomb, out_hbm.at[pl.ds(wid * SPW + h * SPP, SPP)])

    return k


_k = _build()


def kernel(indices, item_table, cat_table, item_to_cat):
    idx3 = indices.reshape(NW, NCHUNK, K)
    return _k(idx3, item_table, cat_table, item_to_cat)


# speedup vs baseline: 5.8329x; 1.1558x over previous
import dataclasses
import functools

import jax
import jax.numpy as jnp
from jax import lax
from jax.experimental import pallas as pl
from jax.experimental.pallas import tpu as pltpu
from jax.experimental.pallas import tpu_sc as plsc

B = 16384
L = 50
D = 64
NCAT = 1000
ALPHA = 0.1

NW = 32
NSUB = 16
TPW = B * L // NW
K = 128
NCHUNK = TPW // K
SPW = B // NW
PASSES = 4
SPP = SPW // PASSES
CPP = NCHUNK // PASSES
NLANE = 16
DEPTH = 5


def _build():
    mesh = plsc.VectorSubcoreMesh(core_axis_name="c", subcore_axis_name="s")
    cp = pltpu.CompilerParams(use_tc_tiling_on_sc=False)
    if "needs_layout_passes" in pltpu.CompilerParams.__dataclass_fields__:
        cp = dataclasses.replace(cp, needs_layout_passes=False)

    scratch = [pltpu.VMEM((CPP, K), jnp.int32)]
    scratch += [pltpu.VMEM((K,), jnp.int32) for _ in range(DEPTH)]
    scratch += [pltpu.VMEM((K,), jnp.int32) for _ in range(DEPTH)]
    scratch += [pltpu.VMEM((K, D), jnp.float32) for _ in range(DEPTH)]
    scratch += [pltpu.VMEM((K, D), jnp.float32) for _ in range(DEPTH)]
    scratch += [
        pltpu.VMEM((SPP, D), jnp.float32),
        pltpu.VMEM((SPP, D), jnp.float32),
        pltpu.VMEM_SHARED((NSUB * SPP, D), jnp.float32),
        pltpu.VMEM_SHARED((NSUB * SPP, D), jnp.float32),
    ]
    scratch += [pltpu.SemaphoreType.DMA] * (4 * DEPTH)

    @functools.partial(
        pl.kernel,
        out_type=jax.ShapeDtypeStruct((B, D), jnp.float32),
        mesh=mesh,
        compiler_params=cp,
        scratch_types=scratch,
    )
    def k(idx_hbm, item_hbm, cat_hbm, i2c_hbm, out_hbm, idx_slab, *rest):
        cidx = rest[0:DEPTH]
        sess = rest[DEPTH:2 * DEPTH]
        ibuf = rest[2 * DEPTH:3 * DEPTH]
        cbuf = rest[3 * DEPTH:4 * DEPTH]
        icomb, ccomb, iacc, cacc = rest[4 * DEPTH:4 * DEPTH + 4]
        sems = rest[4 * DEPTH + 4:]
        semL = sems[0:DEPTH]
        semI = sems[DEPTH:2 * DEPTH]
        semC = sems[2 * DEPTH:3 * DEPTH]
        semS = sems[3 * DEPTH:4 * DEPTH]

        sid = lax.axis_index("s")
        wid = sid * 2 + lax.axis_index("c")

        zero = jnp.zeros((NLANE,), jnp.float32)
        iota = lax.iota(jnp.int32, NLANE)
        inv_l = jnp.float32(1.0 / L)
        alpha = jnp.float32(ALPHA)
        my_rows = pl.ds(sid * SPP, SPP)

        def gather_pair(r, j):
            idx_row = idx_slab.at[r]
            pltpu.async_copy(i2c_hbm.at[idx_row], cidx[j], semL[j])
            pltpu.async_copy(item_hbm.at[idx_row], ibuf[j], semI[j])

        def cat_stage(r, j, h):
            for g in range(K // NLANE):
                tok = iota + ((h * CPP + r) * K + g * NLANE)
                sess[j].at[pl.ds(g * NLANE, NLANE)][...] = (
                    sid * SPP + (tok // L - h * SPP))
            pltpu.make_async_copy(i2c_hbm.at[idx_slab.at[r]], cidx[j],
                                  semL[j]).wait()
            pltpu.async_copy(cat_hbm.at[cidx[j]], cbuf[j], semC[j])

        def scatter_stage(r, j):
            pltpu.make_async_copy(item_hbm.at[idx_slab.at[r]], ibuf[j],
                                  semI[j]).wait()
            pltpu.make_async_copy(cat_hbm.at[cidx[j]], cbuf[j], semC[j]).wait()
            pltpu.async_copy(ibuf[j], iacc.at[sess[j]], semS[j], add=True)
            pltpu.async_copy(cbuf[j], cacc.at[sess[j]], semS[j], add=True)

        def sa_drain(j):
            pltpu.make_async_copy(ibuf[j], iacc.at[sess[j]], semS[j]).wait()
            pltpu.make_async_copy(cbuf[j], cacc.at[sess[j]], semS[j]).wait()

        for h in range(PASSES):
            pltpu.sync_copy(idx_hbm.at[wid, pl.ds(h * CPP, CPP)], idx_slab)

            @pl.loop(0, SPP)
            def _(s):
                for d in range(D // NLANE):
                    icomb.at[s, pl.ds(d * NLANE, NLANE)][...] = zero

            pltpu.sync_copy(icomb, iacc.at[my_rows])
            pltpu.sync_copy(icomb, cacc.at[my_rows])

            gather_pair(0, 0)
            gather_pair(1, 1)
            cat_stage(0, 0, h)

            @pl.loop(0, CPP // DEPTH)
            def _(t):
                for j in range(DEPTH):
                    r = DEPTH * t + j
                    sj2 = (j + 2) % DEPTH
                    sj1 = (j + 1) % DEPTH

                    @pl.when(r >= DEPTH - 2)
                    def _(sj2=sj2):
                        sa_drain(sj2)

                    @pl.when(r <= CPP - 3)
                    def _(r=r, sj2=sj2):
                        gather_pair(r + 2, sj2)

                    @pl.when(r <= CPP - 2)
                    def _(r=r, sj1=sj1):
                        cat_stage(r + 1, sj1, h)

                    scatter_stage(r, j)

            for tail in range(DEPTH - 2, 0, -1):
                sa_drain((CPP - tail) % DEPTH)

            pltpu.sync_copy(iacc.at[my_rows], icomb)
            pltpu.sync_copy(cacc.at[my_rows], ccomb)

            @pl.loop(0, SPP)
            def _(s):
                for d in range(D // NLANE):
                    slc = (s, pl.ds(d * NLANE, NLANE))
                    a = icomb.at[slc][...]
                    b = ccomb.at[slc][...]
                    icomb.at[slc][...] = (a + alpha * b) * inv_l

            pltpu.sync_copy(icomb, out_hbm.at[pl.ds(wid * SPW + h * SPP, SPP)])

    return k


_k = _build()


def kernel(indices, item_table, cat_table, item_to_cat):
    idx3 = indices.reshape(NW, NCHUNK, K)
    return _k(idx3, item_table, cat_table, item_to_cat)
